# baseline (speedup 1.0000x reference)
import jax
from jax.experimental import pallas as pl
from jax.experimental.pallas import tpu as pltpu


def _identity_copy_kernel(x_ref, o_ref):
    def body(sem):
        cp = pltpu.make_async_copy(x_ref, o_ref, sem)
        cp.start()
        cp.wait()

    pl.run_scoped(body, pltpu.SemaphoreType.DMA)


def kernel(x):
    return pl.pallas_call(
        _identity_copy_kernel,
        in_specs=[pl.BlockSpec(memory_space=pl.ANY)],
        out_specs=pl.BlockSpec(memory_space=pl.ANY),
        out_shape=jax.ShapeDtypeStruct(x.shape, x.dtype),
    )(x)
